# Initial kernel scaffold; baseline (speedup 1.0000x reference)
import jax
import jax.numpy as jnp
from jax.experimental import pallas as pl


def kernel(x, edge_index, W_emb, b_emb, conv_w, conv_b, bn_g, bn_b, W_out, b_out):
    raise NotImplementedError("write your pallas kernel here")


# speedup vs baseline: 6.7794x; 6.7794x over previous
import functools

import jax
import jax.numpy as jnp
from jax import lax
from jax.experimental import pallas as pl
from jax.experimental.pallas import tpu as pltpu
from jax.experimental.pallas import tpu_sc as plsc

N = 10000
D = 128
NPAD = 10112
NW = 32
K = 80
B = 128
G = 16
EP = NW * K * B
STRIPE = NPAD // 16

_MESH = plsc.VectorSubcoreMesh(core_axis_name="c", subcore_axis_name="s")


@functools.partial(
    pl.kernel,
    out_type=jax.ShapeDtypeStruct((2, NPAD, D), jnp.float32),
    mesh=_MESH,
    scratch_types=[
        pltpu.VMEM((K, B), jnp.int32),
        pltpu.VMEM((B, D), jnp.float32),
        pltpu.VMEM_SHARED((NPAD, D), jnp.float32),
    ],
)
def _deg_kernel(dst_hbm, ones_hbm, zeros_hbm, out_hbm, dst_v, ones_v, acc):
    cid = lax.axis_index("c")
    sid = lax.axis_index("s")
    wid = cid * 16 + sid
    pltpu.sync_copy(dst_hbm.at[wid], dst_v)
    pltpu.sync_copy(ones_hbm, ones_v)
    pltpu.sync_copy(zeros_hbm, acc.at[pl.ds(sid * STRIPE, STRIPE)])
    plsc.subcore_barrier()

    @pl.loop(0, K)
    def _(c):
        pltpu.sync_copy(ones_v, acc.at[dst_v.at[c]], add=True)

    plsc.subcore_barrier()
    pltpu.sync_copy(acc.at[pl.ds(sid * STRIPE, STRIPE)],
                    out_hbm.at[cid, pl.ds(sid * STRIPE, STRIPE)])


@functools.partial(
    pl.kernel,
    out_type=jax.ShapeDtypeStruct((2, NPAD, D), jnp.float32),
    mesh=_MESH,
    scratch_types=[
        pltpu.VMEM((G, B), jnp.int32),
        pltpu.VMEM((G, B), jnp.int32),
        pltpu.VMEM((B, D), jnp.float32),
        pltpu.VMEM((B, D), jnp.float32),
        pltpu.VMEM_SHARED((NPAD, D), jnp.float32),
        pltpu.SemaphoreType.DMA,
        pltpu.SemaphoreType.DMA,
    ],
)
def _scatter_kernel(y_hbm, src_hbm, dst_hbm, zeros_hbm, out_hbm,
                    src_v, dst_v, rows0, rows1, acc, sem0, sem1):
    cid = lax.axis_index("c")
    sid = lax.axis_index("s")
    wid = cid * 16 + sid
    pltpu.sync_copy(zeros_hbm, acc.at[pl.ds(sid * STRIPE, STRIPE)])
    plsc.subcore_barrier()

    rows = (rows0, rows1)
    sems = (sem0, sem1)
    for grp in range(K // G):
        pltpu.sync_copy(src_hbm.at[wid, pl.ds(grp * G, G)], src_v)
        pltpu.sync_copy(dst_hbm.at[wid, pl.ds(grp * G, G)], dst_v)
        pltpu.async_copy(y_hbm.at[src_v.at[0]], rows0, sem0)
        pltpu.async_copy(y_hbm.at[src_v.at[1]], rows1, sem1)

        @pl.loop(0, G - 2, step=2)
        def _(g):
            for b in range(2):
                c = g + b
                pltpu.make_async_copy(y_hbm.at[src_v.at[c]], rows[b],
                                      sems[b]).wait()
                pltpu.sync_copy(rows[b], acc.at[dst_v.at[c]], add=True)
                pltpu.async_copy(y_hbm.at[src_v.at[c + 2]], rows[b], sems[b])

        for b in range(2):
            c = G - 2 + b
            pltpu.make_async_copy(y_hbm.at[src_v.at[c]], rows[b],
                                  sems[b]).wait()
            pltpu.sync_copy(rows[b], acc.at[dst_v.at[c]], add=True)

    plsc.subcore_barrier()
    pltpu.sync_copy(acc.at[pl.ds(sid * STRIPE, STRIPE)],
                    out_hbm.at[cid, pl.ds(sid * STRIPE, STRIPE)])


def _deg_inv(degp_ref):
    deg = degp_ref[0, :N, 0:1] + degp_ref[1, :N, 0:1] + 1.0
    return lax.rsqrt(deg)


def _emb_body(x_ref, wemb_ref, bemb_ref, w0_ref, degp_ref, y0_ref):
    h = jnp.dot(x_ref[...], wemb_ref[...],
                preferred_element_type=jnp.float32) + bemb_ref[...]
    xw = jnp.dot(h, w0_ref[...], preferred_element_type=jnp.float32)
    y0_ref[...] = xw * _deg_inv(degp_ref)


def _bn_relu(aggp_ref, y_ref, degp_ref, cb_ref, g_ref, bb_ref):
    dinv = _deg_inv(degp_ref)
    t = dinv * (aggp_ref[0, :N, :] + aggp_ref[1, :N, :] + y_ref[...]) + cb_ref[...]
    mu = jnp.mean(t, axis=0, keepdims=True)
    dev = t - mu
    var = jnp.mean(dev * dev, axis=0, keepdims=True)
    h = jnp.maximum(dev * lax.rsqrt(var + 1e-5) * g_ref[...] + bb_ref[...], 0.0)
    return h, dinv


def _layer_body(aggp_ref, y_ref, degp_ref, cb_ref, g_ref, bb_ref, wn_ref, out_ref):
    h, dinv = _bn_relu(aggp_ref, y_ref, degp_ref, cb_ref, g_ref, bb_ref)
    out_ref[...] = jnp.dot(dinv * h, wn_ref[...],
                           preferred_element_type=jnp.float32)


def _last_body(aggp_ref, y_ref, degp_ref, cb_ref, g_ref, bb_ref,
               wout_ref, bout_ref, out_ref):
    h, _ = _bn_relu(aggp_ref, y_ref, degp_ref, cb_ref, g_ref, bb_ref)
    s = jnp.sum(h, axis=0, keepdims=True) * (1.0 / N)
    out_ref[...] = jnp.dot(s, wout_ref[...],
                           preferred_element_type=jnp.float32) + bout_ref[...]


def kernel(x, edge_index, W_emb, b_emb, conv_w, conv_b, bn_g, bn_b, W_out, b_out):
    src = edge_index[0]
    dst = edge_index[1]
    pad = EP - src.shape[0]
    src_t = jnp.concatenate([src, jnp.zeros((pad,), jnp.int32)]).reshape(NW, K, B)
    dst_t = jnp.concatenate([dst, jnp.full((pad,), N, jnp.int32)]).reshape(NW, K, B)
    onesD = jnp.ones((B, D), jnp.float32)
    zerosD = jnp.zeros((STRIPE, D), jnp.float32)

    degp = _deg_kernel(dst_t, onesD, zerosD)

    y = pl.pallas_call(
        _emb_body, out_shape=jax.ShapeDtypeStruct((N, D), jnp.float32))(
            x, W_emb, b_emb.reshape(1, D), conv_w[0], degp)

    out = None
    for l in range(4):
        aggp = _scatter_kernel(y, src_t, dst_t, zerosD)
        if l < 3:
            y = pl.pallas_call(
                _layer_body, out_shape=jax.ShapeDtypeStruct((N, D), jnp.float32))(
                    aggp, y, degp, conv_b[l].reshape(1, D), bn_g[l].reshape(1, D),
                    bn_b[l].reshape(1, D), conv_w[l + 1])
        else:
            out = pl.pallas_call(
                _last_body, out_shape=jax.ShapeDtypeStruct((1, 1), jnp.float32))(
                    aggp, y, degp, conv_b[l].reshape(1, D), bn_g[l].reshape(1, D),
                    bn_b[l].reshape(1, D), W_out, b_out.reshape(1, 1))
    return out


# speedup vs baseline: 6.9108x; 1.0194x over previous
import functools

import jax
import jax.numpy as jnp
from jax import lax
from jax.experimental import pallas as pl
from jax.experimental.pallas import tpu as pltpu
from jax.experimental.pallas import tpu_sc as plsc

N = 10000
D = 128
NPAD = 10112
NW = 32
K = 80
B = 128
G = 16
EP = NW * K * B
KF = 128
KS = 32
FAST = 0
ROWS = 16 * (KF + KS)
STRIPE = NPAD // 16

_MESH = plsc.VectorSubcoreMesh(core_axis_name="c", subcore_axis_name="s")


@functools.partial(
    pl.kernel,
    out_type=jax.ShapeDtypeStruct((2, NPAD, D), jnp.float32),
    mesh=_MESH,
    scratch_types=[
        pltpu.VMEM((K, B), jnp.int32),
        pltpu.VMEM((B, D), jnp.float32),
        pltpu.VMEM_SHARED((NPAD, D), jnp.float32),
    ],
)
def _deg_kernel(dst_hbm, ones_hbm, zeros_hbm, out_hbm, dst_v, ones_v, acc):
    cid = lax.axis_index("c")
    sid = lax.axis_index("s")
    wid = cid * 16 + sid
    pltpu.sync_copy(dst_hbm.at[wid], dst_v)
    pltpu.sync_copy(ones_hbm, ones_v)
    pltpu.sync_copy(zeros_hbm, acc.at[pl.ds(sid * STRIPE, STRIPE)])
    plsc.subcore_barrier()

    @pl.loop(0, K)
    def _(c):
        pltpu.sync_copy(ones_v, acc.at[dst_v.at[c]], add=True)

    plsc.subcore_barrier()
    pltpu.sync_copy(acc.at[pl.ds(sid * STRIPE, STRIPE)],
                    out_hbm.at[cid, pl.ds(sid * STRIPE, STRIPE)])


@functools.partial(
    pl.kernel,
    out_type=jax.ShapeDtypeStruct((2, NPAD, D), jnp.float32),
    mesh=_MESH,
    scratch_types=[
        pltpu.VMEM((G, B), jnp.int32),
        pltpu.VMEM((G, B), jnp.int32),
        pltpu.VMEM((B, D), jnp.float32),
        pltpu.VMEM((B, D), jnp.float32),
        pltpu.VMEM_SHARED((NPAD, D), jnp.float32),
        pltpu.SemaphoreType.DMA,
        pltpu.SemaphoreType.DMA,
    ],
)
def _scatter_kernel(y_hbm, src_hbm, dst_hbm, zeros_hbm, out_hbm,
                    src_v, dst_v, rows0, rows1, acc, sem0, sem1):
    cid = lax.axis_index("c")
    sid = lax.axis_index("s")
    fast = cid == FAST
    base = jnp.where(fast, sid * KF, 16 * KF + sid * KS)
    n_grp = jnp.where(fast, KF // G, KS // G)
    pltpu.sync_copy(zeros_hbm, acc.at[pl.ds(sid * STRIPE, STRIPE)])
    plsc.subcore_barrier()

    rows = (rows0, rows1)
    sems = (sem0, sem1)

    @pl.loop(0, n_grp)
    def _(grp):
        pltpu.sync_copy(src_hbm.at[pl.ds(base + grp * G, G)], src_v)
        pltpu.sync_copy(dst_hbm.at[pl.ds(base + grp * G, G)], dst_v)
        pltpu.async_copy(y_hbm.at[src_v.at[0]], rows0, sem0)
        pltpu.async_copy(y_hbm.at[src_v.at[1]], rows1, sem1)

        @pl.loop(0, G - 2, step=2)
        def _(g):
            for b in range(2):
                c = g + b
                pltpu.make_async_copy(y_hbm.at[src_v.at[c]], rows[b],
                                      sems[b]).wait()
                pltpu.sync_copy(rows[b], acc.at[dst_v.at[c]], add=True)
                pltpu.async_copy(y_hbm.at[src_v.at[c + 2]], rows[b], sems[b])

        for b in range(2):
            c = G - 2 + b
            pltpu.make_async_copy(y_hbm.at[src_v.at[c]], rows[b],
                                  sems[b]).wait()
            pltpu.sync_copy(rows[b], acc.at[dst_v.at[c]], add=True)

    plsc.subcore_barrier()
    pltpu.sync_copy(acc.at[pl.ds(sid * STRIPE, STRIPE)],
                    out_hbm.at[cid, pl.ds(sid * STRIPE, STRIPE)])


def _deg_inv(degp_ref):
    deg = degp_ref[0, :N, 0:1] + degp_ref[1, :N, 0:1] + 1.0
    return lax.rsqrt(deg)


def _emb_body(x_ref, wemb_ref, bemb_ref, w0_ref, degp_ref, y0_ref):
    h = jnp.dot(x_ref[...], wemb_ref[...],
                preferred_element_type=jnp.float32) + bemb_ref[...]
    xw = jnp.dot(h, w0_ref[...], preferred_element_type=jnp.float32)
    y0_ref[...] = xw * _deg_inv(degp_ref)


def _bn_relu(aggp_ref, y_ref, degp_ref, cb_ref, g_ref, bb_ref):
    dinv = _deg_inv(degp_ref)
    t = dinv * (aggp_ref[0, :N, :] + aggp_ref[1, :N, :] + y_ref[...]) + cb_ref[...]
    mu = jnp.mean(t, axis=0, keepdims=True)
    dev = t - mu
    var = jnp.mean(dev * dev, axis=0, keepdims=True)
    h = jnp.maximum(dev * lax.rsqrt(var + 1e-5) * g_ref[...] + bb_ref[...], 0.0)
    return h, dinv


def _layer_body(aggp_ref, y_ref, degp_ref, cb_ref, g_ref, bb_ref, wn_ref, out_ref):
    h, dinv = _bn_relu(aggp_ref, y_ref, degp_ref, cb_ref, g_ref, bb_ref)
    out_ref[...] = jnp.dot(dinv * h, wn_ref[...],
                           preferred_element_type=jnp.float32)


def _last_body(aggp_ref, y_ref, degp_ref, cb_ref, g_ref, bb_ref,
               wout_ref, bout_ref, out_ref):
    h, _ = _bn_relu(aggp_ref, y_ref, degp_ref, cb_ref, g_ref, bb_ref)
    s = jnp.sum(h, axis=0, keepdims=True) * (1.0 / N)
    out_ref[...] = jnp.dot(s, wout_ref[...],
                           preferred_element_type=jnp.float32) + bout_ref[...]


def kernel(x, edge_index, W_emb, b_emb, conv_w, conv_b, bn_g, bn_b, W_out, b_out):
    src = edge_index[0]
    dst = edge_index[1]
    pad = EP - src.shape[0]
    src_p = jnp.concatenate([src, jnp.zeros((pad,), jnp.int32)])
    dst_p = jnp.concatenate([dst, jnp.full((pad,), N, jnp.int32)])
    dst_t = dst_p.reshape(NW, K, B)
    src_f = src_p.reshape(ROWS, B)
    dst_f = dst_p.reshape(ROWS, B)
    onesD = jnp.ones((B, D), jnp.float32)
    zerosD = jnp.zeros((STRIPE, D), jnp.float32)

    degp = _deg_kernel(dst_t, onesD, zerosD)

    y = pl.pallas_call(
        _emb_body, out_shape=jax.ShapeDtypeStruct((N, D), jnp.float32))(
            x, W_emb, b_emb.reshape(1, D), conv_w[0], degp)

    out = None
    for l in range(4):
        aggp = _scatter_kernel(y, src_f, dst_f, zerosD)
        if l < 3:
            y = pl.pallas_call(
                _layer_body, out_shape=jax.ShapeDtypeStruct((N, D), jnp.float32))(
                    aggp, y, degp, conv_b[l].reshape(1, D), bn_g[l].reshape(1, D),
                    bn_b[l].reshape(1, D), conv_w[l + 1])
        else:
            out = pl.pallas_call(
                _last_body, out_shape=jax.ShapeDtypeStruct((1, 1), jnp.float32))(
                    aggp, y, degp, conv_b[l].reshape(1, D), bn_g[l].reshape(1, D),
                    bn_b[l].reshape(1, D), W_out, b_out.reshape(1, 1))
    return out
